# Initial kernel scaffold; baseline (speedup 1.0000x reference)
import jax
import jax.numpy as jnp
from jax.experimental import pallas as pl


def kernel(t, pe):
    raise NotImplementedError("write your pallas kernel here")


# speedup vs baseline: 1.3829x; 1.3829x over previous
import functools

import jax
import jax.numpy as jnp
from jax import lax
from jax.experimental import pallas as pl
from jax.experimental.pallas import tpu as pltpu
from jax.experimental.pallas import tpu_sc as plsc

N_EMBD = 320
TIME_STEPS = 1000
BATCH = 16384

_NC = 2
_NS = 16
_NW = _NC * _NS
_B_PER_W = BATCH // _NW
_CHUNK = 128
_N_CHUNKS = _B_PER_W // _CHUNK


def _make_gather():
    mesh = plsc.VectorSubcoreMesh(core_axis_name="c", subcore_axis_name="s")

    @functools.partial(
        pl.kernel,
        mesh=mesh,
        compiler_params=pltpu.CompilerParams(use_tc_tiling_on_sc=False),
        out_type=jax.ShapeDtypeStruct((BATCH, N_EMBD), jnp.float32),
        scratch_types=[
            pltpu.VMEM((_N_CHUNKS, _CHUNK), jnp.int32),
            pltpu.VMEM((_CHUNK, N_EMBD), jnp.float32),
            pltpu.VMEM((_CHUNK, N_EMBD), jnp.float32),
            pltpu.SemaphoreType.DMA,
            pltpu.SemaphoreType.DMA,
        ],
    )
    def k(t_hbm, pe_hbm, out_hbm, idx_v, buf0, buf1, sem0, sem1):
        wid = lax.axis_index("s") * _NC + lax.axis_index("c")
        base = wid * _B_PER_W
        pltpu.sync_copy(t_hbm.at[wid], idx_v)

        bufs = (buf0, buf1)
        sems = (sem0, sem1)
        copies = [None] * _N_CHUNKS
        copies[0] = pltpu.async_copy(pe_hbm.at[idx_v.at[0]], buf0, sem0)
        for c in range(_N_CHUNKS):
            if c + 1 < _N_CHUNKS:
                copies[c + 1] = pltpu.async_copy(
                    pe_hbm.at[idx_v.at[c + 1]], bufs[(c + 1) % 2], sems[(c + 1) % 2]
                )
            copies[c].wait()
            pltpu.sync_copy(
                bufs[c % 2], out_hbm.at[pl.ds(base + c * _CHUNK, _CHUNK)]
            )

    return k


_gather = _make_gather()


@jax.jit
def kernel(t, pe):
    t3 = t.astype(jnp.int32).reshape(_NW, _N_CHUNKS, _CHUNK)
    return _gather(t3, pe)


# speedup vs baseline: 1.3877x; 1.0034x over previous
import functools

import jax
import jax.numpy as jnp
from jax import lax
from jax.experimental import pallas as pl
from jax.experimental.pallas import tpu as pltpu
from jax.experimental.pallas import tpu_sc as plsc

N_EMBD = 320
TIME_STEPS = 1000
BATCH = 16384

_NC = 2
_NS = 16
_NW = _NC * _NS
_B_PER_W = BATCH // _NW
_CHUNK = 64
_N_CHUNKS = _B_PER_W // _CHUNK
_NBUF = 6


def _make_gather():
    mesh = plsc.VectorSubcoreMesh(core_axis_name="c", subcore_axis_name="s")

    @functools.partial(
        pl.kernel,
        mesh=mesh,
        compiler_params=pltpu.CompilerParams(use_tc_tiling_on_sc=False),
        out_type=jax.ShapeDtypeStruct((BATCH, N_EMBD), jnp.float32),
        scratch_types=[
            pltpu.VMEM((_N_CHUNKS, _CHUNK), jnp.int32),
        ]
        + [pltpu.VMEM((_CHUNK, N_EMBD), jnp.float32) for _ in range(_NBUF)]
        + [pltpu.SemaphoreType.DMA for _ in range(2 * _NBUF)],
    )
    def k(t_hbm, pe_hbm, out_hbm, idx_v, *rest):
        bufs = rest[:_NBUF]
        gsems = rest[_NBUF : 2 * _NBUF]
        ssems = rest[2 * _NBUF :]
        wid = lax.axis_index("s") * _NC + lax.axis_index("c")
        base = wid * _B_PER_W
        pltpu.sync_copy(t_hbm.at[wid], idx_v)

        g = [None] * _N_CHUNKS
        s = [None] * _N_CHUNKS
        for c in range(min(_NBUF, _N_CHUNKS)):
            g[c] = pltpu.async_copy(pe_hbm.at[idx_v.at[c]], bufs[c], gsems[c])
        for c in range(_N_CHUNKS):
            b = c % _NBUF
            g[c].wait()
            s[c] = pltpu.async_copy(
                bufs[b], out_hbm.at[pl.ds(base + c * _CHUNK, _CHUNK)], ssems[b]
            )
            n = c + _NBUF
            if n < _N_CHUNKS:
                s[n - _NBUF].wait()
                g[n] = pltpu.async_copy(
                    pe_hbm.at[idx_v.at[n]], bufs[n % _NBUF], gsems[n % _NBUF]
                )
        for c in range(max(0, _N_CHUNKS - _NBUF), _N_CHUNKS):
            s[c].wait()

    return k


_gather = _make_gather()


@jax.jit
def kernel(t, pe):
    t3 = t.astype(jnp.int32).reshape(_NW, _N_CHUNKS, _CHUNK)
    return _gather(t3, pe)


# speedup vs baseline: 1.8762x; 1.3520x over previous
import functools

import jax
import jax.numpy as jnp
from jax import lax
from jax.experimental import pallas as pl
from jax.experimental.pallas import tpu as pltpu
from jax.experimental.pallas import tpu_sc as plsc

N_EMBD = 320
TIME_STEPS = 1000
BATCH = 16384

_D_PAD = 384
_NC = 2
_NS = 16
_NW = _NC * _NS
_B_PER_W = BATCH // _NW
_CHUNK = 128
_N_CHUNKS = _B_PER_W // _CHUNK
_NBUF = 2


def _make_gather():
    mesh = plsc.VectorSubcoreMesh(core_axis_name="c", subcore_axis_name="s")

    @functools.partial(
        pl.kernel,
        mesh=mesh,
        out_type=jax.ShapeDtypeStruct((BATCH, _D_PAD), jnp.float32),
        scratch_types=[
            pltpu.VMEM((_N_CHUNKS, _CHUNK), jnp.int32),
        ]
        + [pltpu.VMEM((_CHUNK, _D_PAD), jnp.float32) for _ in range(_NBUF)]
        + [pltpu.SemaphoreType.DMA for _ in range(2 * _NBUF)],
    )
    def k(t_hbm, pe_hbm, out_hbm, idx_v, *rest):
        bufs = rest[:_NBUF]
        gsems = rest[_NBUF : 2 * _NBUF]
        ssems = rest[2 * _NBUF :]
        wid = lax.axis_index("s") * _NC + lax.axis_index("c")
        base = wid * _B_PER_W
        pltpu.sync_copy(t_hbm.at[wid], idx_v)

        g = [None] * _N_CHUNKS
        s = [None] * _N_CHUNKS
        for c in range(min(_NBUF, _N_CHUNKS)):
            g[c] = pltpu.async_copy(pe_hbm.at[idx_v.at[c]], bufs[c], gsems[c])
        for c in range(_N_CHUNKS):
            b = c % _NBUF
            g[c].wait()
            s[c] = pltpu.async_copy(
                bufs[b], out_hbm.at[pl.ds(base + c * _CHUNK, _CHUNK)], ssems[b]
            )
            n = c + _NBUF
            if n < _N_CHUNKS:
                s[n - _NBUF].wait()
                g[n] = pltpu.async_copy(
                    pe_hbm.at[idx_v.at[n]], bufs[n % _NBUF], gsems[n % _NBUF]
                )
        for c in range(max(0, _N_CHUNKS - _NBUF), _N_CHUNKS):
            s[c].wait()

    return k


_gather = _make_gather()


@jax.jit
def kernel(t, pe):
    t3 = t.astype(jnp.int32).reshape(_NW, _N_CHUNKS, _CHUNK)
    pe_pad = jnp.pad(pe, ((0, 0), (0, _D_PAD - N_EMBD)))
    return _gather(t3, pe_pad)[:, :N_EMBD]
